# baseline (speedup 1.0000x reference)
import jax
import jax.numpy as jnp
from jax import lax
from jax.experimental import pallas as pl
from jax.experimental.pallas import tpu as pltpu

B = 16
N = 8192
M = 1024


def _fps_kernel(x_ref, y_ref, z_ref, cx_ref, cy_ref, cz_ref, d_ref):
    d_ref[...] = jnp.full((B, N), jnp.inf, dtype=jnp.float32)
    iota = lax.broadcasted_iota(jnp.int32, (B, N), 1)

    def body(k, carry):
        fx, fy, fz = carry
        cx_ref[pl.ds(k, 1), :] = fx.reshape(1, B)
        cy_ref[pl.ds(k, 1), :] = fy.reshape(1, B)
        cz_ref[pl.ds(k, 1), :] = fz.reshape(1, B)

        dx = x_ref[...] - fx
        dy = y_ref[...] - fy
        dz = z_ref[...] - fz
        dist = (dx * dx + dz * dz) + dy * dy
        d = jnp.minimum(d_ref[...], dist)
        d_ref[...] = d

        m = jnp.max(d, axis=1, keepdims=True)
        cand = jnp.where(d == m, iota, N)
        j = jnp.min(cand, axis=1, keepdims=True)
        onehot = iota == j
        ow, xw, yw, zw = onehot, x_ref[...], y_ref[...], z_ref[...]
        w = N
        while w > 1:
            h = w // 2
            oa, ob = ow[:, :h], ow[:, h:w]
            xw = jnp.where(oa, xw[:, :h], xw[:, h:w])
            yw = jnp.where(oa, yw[:, :h], yw[:, h:w])
            zw = jnp.where(oa, zw[:, :h], zw[:, h:w])
            ow = oa | ob
            w = h
        return xw, yw, zw

    init = (x_ref[:, 0:1], y_ref[:, 0:1], z_ref[:, 0:1])
    lax.fori_loop(0, M, body, init)


@jax.jit
def kernel(xyz):
    x = xyz[:, :, 0]
    y = xyz[:, :, 1]
    z = xyz[:, :, 2]
    out_shape = jax.ShapeDtypeStruct((M, B), jnp.float32)
    cx, cy, cz = pl.pallas_call(
        _fps_kernel,
        out_shape=(out_shape, out_shape, out_shape),
        scratch_shapes=[pltpu.VMEM((B, N), jnp.float32)],
    )(x, y, z)
    return jnp.stack([cx.T, cy.T, cz.T], axis=-1)


# speedup vs baseline: 1.5753x; 1.5753x over previous
import jax
import jax.numpy as jnp
from jax import lax
from jax.experimental import pallas as pl
from jax.experimental.pallas import tpu as pltpu

B = 16
N = 8192
M = 1024


def _fps_kernel(x_ref, y_ref, z_ref, c_ref):
    x = x_ref[...]
    y = y_ref[...]
    z = z_ref[...]
    iota = lax.broadcasted_iota(jnp.int32, (B, N), 1)
    zero = jnp.zeros((B, N), dtype=jnp.float32)

    def body(k, carry):
        d_prev, fx, fy, fz = carry
        c_ref[pl.ds(k, 1), :] = jnp.concatenate(
            [fx.reshape(1, B), fy.reshape(1, B), fz.reshape(1, B)], axis=1
        )

        dx = x - fx
        dy = y - fy
        dz = z - fz
        dist = (dx * dx + dz * dz) + dy * dy
        d = jnp.minimum(d_prev, dist)

        m = jnp.max(d, axis=1, keepdims=True)
        cand = jnp.where(d == m, iota, N)
        j = jnp.min(cand, axis=1, keepdims=True)
        onehot = iota == j
        nfx = jnp.sum(jnp.where(onehot, x, zero), axis=1, keepdims=True)
        nfy = jnp.sum(jnp.where(onehot, y, zero), axis=1, keepdims=True)
        nfz = jnp.sum(jnp.where(onehot, z, zero), axis=1, keepdims=True)
        return d, nfx, nfy, nfz

    init = (
        jnp.full((B, N), jnp.inf, dtype=jnp.float32),
        x[:, 0:1],
        y[:, 0:1],
        z[:, 0:1],
    )
    lax.fori_loop(0, M, body, init)


@jax.jit
def kernel(xyz):
    x = xyz[:, :, 0]
    y = xyz[:, :, 1]
    z = xyz[:, :, 2]
    c = pl.pallas_call(
        _fps_kernel,
        out_shape=jax.ShapeDtypeStruct((M, 3 * B), jnp.float32),
    )(x, y, z)
    return jnp.stack([c[:, :B].T, c[:, B : 2 * B].T, c[:, 2 * B :].T], axis=-1)
